# Initial kernel scaffold; baseline (speedup 1.0000x reference)
import jax
import jax.numpy as jnp
from jax.experimental import pallas as pl


def kernel(h, edge_index_0, edge_index_1, edge_index_2, W_0, W_1, W_2, attn_l_0, attn_l_1, attn_l_2, attn_r_0, attn_r_1, attn_r_2, sem_W1, sem_b1, sem_W2):
    raise NotImplementedError("write your pallas kernel here")


# speedup vs baseline: 37.8264x; 37.8264x over previous
import functools

import jax
import jax.numpy as jnp
from jax import lax
from jax.experimental import pallas as pl
from jax.experimental.pallas import tpu as pltpu
from jax.experimental.pallas import tpu_sc as plsc

N = 100000
E = 1600000
IN = 32
D = 32
HID = 128
NP = 100096
CH = NP // 16
GSZ = 512
NG = E // GSZ
KMAX = (NG + 15) // 16
BN = 1000
NB = N // BN


def _k1_body(h_ref, w3_ref, al3_ref, ar3_ref, f3_ref, el3_ref, er3_ref):
    hb = h_ref[...]
    for i in range(3):
        f = jnp.dot(hb, w3_ref[i], preferred_element_type=jnp.float32)
        el = jnp.sum(f * al3_ref[i], axis=1, keepdims=True)
        er = jnp.sum(f * ar3_ref[i], axis=1, keepdims=True)
        f3_ref[i, 0] = jnp.concatenate([f[:, :16], el], axis=1)
        f3_ref[i, 1] = jnp.concatenate([f[:, 16:], el], axis=1)
        el3_ref[i] = el
        er3_ref[i] = er


def _k1(h, w3, al3, ar3):
    return pl.pallas_call(
        _k1_body,
        grid=(NB,),
        in_specs=[
            pl.BlockSpec((BN, IN), lambda b: (b, 0)),
            pl.BlockSpec((3, IN, D), lambda b: (0, 0, 0)),
            pl.BlockSpec((3, 1, D), lambda b: (0, 0, 0)),
            pl.BlockSpec((3, 1, D), lambda b: (0, 0, 0)),
        ],
        out_specs=[
            pl.BlockSpec((3, 2, BN, 17), lambda b: (0, 0, b, 0)),
            pl.BlockSpec((3, BN, 1), lambda b: (0, b, 0)),
            pl.BlockSpec((3, BN, 1), lambda b: (0, b, 0)),
        ],
        out_shape=[
            jax.ShapeDtypeStruct((3, 2, N, 17), jnp.float32),
            jax.ShapeDtypeStruct((3, N, 1), jnp.float32),
            jax.ShapeDtypeStruct((3, N, 1), jnp.float32),
        ],
    )(h, w3, al3, ar3)


_mesh = plsc.VectorSubcoreMesh(core_axis_name="c", subcore_axis_name="s")


@functools.partial(
    pl.kernel,
    out_type=[
        jax.ShapeDtypeStruct((NP, 16), jnp.float32),
        jax.ShapeDtypeStruct((NP, 16), jnp.float32),
        jax.ShapeDtypeStruct((NP,), jnp.float32),
    ],
    mesh=_mesh,
    compiler_params=pltpu.CompilerParams(use_tc_tiling_on_sc=False,
                                         needs_layout_passes=False),
    scratch_types=[
        pltpu.VMEM_SHARED((NP,), jnp.float32),
        pltpu.VMEM_SHARED((NP, 16), jnp.float32),
        pltpu.VMEM((4, 128), jnp.int32),
        pltpu.VMEM((4, 128), jnp.int32),
        pltpu.VMEM((4, 128), jnp.int32),
        pltpu.VMEM((GSZ,), jnp.float32),
        pltpu.VMEM((GSZ,), jnp.float32),
        pltpu.VMEM((GSZ, 17), jnp.float32),
        pltpu.VMEM((GSZ, 16), jnp.float32),
        pltpu.VMEM((16,), jnp.float32),
        pltpu.SemaphoreType.DMA,
    ],
)
def _edge_pass(featT, erp, src2d, dst2d, csh,
               rstL, rstR, esum_out,
               esum_s, rst_s,
               srcv, dstv, fidx, erdv, exv, rows, msg, cv, sem):
    c = lax.axis_index("c")
    s = lax.axis_index("s")
    zeros16 = jnp.zeros((16,), jnp.float32)
    off = s * CH
    _chunks = [(q * GSZ, GSZ) for q in range(12)] + [(12 * GSZ, 112)]

    def _zm(i, carry):
        msg[i, :] = zeros16
        return carry

    lax.fori_loop(0, GSZ, _zm, 0)

    def _zx(i, carry):
        exv[pl.ds(i * 16, 16)] = zeros16
        return carry

    lax.fori_loop(0, GSZ // 16, _zx, 0)

    for qo, sz in _chunks:
        pltpu.sync_copy(exv.at[pl.ds(0, sz)], esum_s.at[pl.ds(off + qo, sz)])
        pltpu.sync_copy(msg.at[pl.ds(0, sz), :],
                        rst_s.at[pl.ds(off + qo, sz), :])
    pltpu.sync_copy(csh, cv)
    plsc.subcore_barrier()

    cN = c * N
    cvec = cv[...]
    tv = [jnp.full((16,), t, jnp.int32) for t in range(16)]
    col16 = jnp.full((16,), 16, jnp.int32)

    def _group(k, carry):
        g = k * 16 + s

        @pl.when(g < NG)
        def _do():
            pltpu.sync_copy(src2d.at[pl.ds(g * 4, 4), :], srcv)
            pltpu.sync_copy(dst2d.at[pl.ds(g * 4, 4), :], dstv)
            for q in range(4):
                for r in range(0, 128, 16):
                    slq = pl.ds(r, 16)
                    fidx[q, slq] = srcv[q, slq] + cN
            cps = [pltpu.async_copy(featT.at[fidx.at[q]],
                                    rows.at[pl.ds(q * 128, 128), :], sem)
                   for q in range(4)]
            cps += [pltpu.async_copy(erp.at[dstv.at[q]],
                                     erdv.at[pl.ds(q * 128, 128)], sem)
                    for q in range(4)]
            for cp in cps:
                cp.wait()
            for j in range(GSZ // 16):
                sl = pl.ds(j * 16, 16)
                el16 = plsc.load_gather(rows, [lax.iota(jnp.int32, 16)
                                               + (j * 16), col16])
                e16 = el16 + erdv[sl]
                lk = jnp.maximum(e16, 0.2 * e16)
                ex16 = jnp.exp(lk - cvec)
                exv[sl] = ex16
                for t in range(16):
                    spl = ex16.at[tv[t]].get(mode="promise_in_bounds")
                    e = j * 16 + t
                    msg[e, :] = rows[e, pl.ds(0, 16)] * spl
            for q in range(4):
                pltpu.sync_copy(msg.at[pl.ds(q * 128, 128), :],
                                rst_s.at[dstv.at[q]], add=True)

            @pl.when(c == 0)
            def _esum():
                for q in range(4):
                    pltpu.sync_copy(exv.at[pl.ds(q * 128, 128)],
                                    esum_s.at[dstv.at[q]], add=True)

        return carry

    lax.fori_loop(0, KMAX, _group, 0)
    plsc.subcore_barrier()

    @pl.when(c == 0)
    def _w0():
        for qo, sz in _chunks:
            pltpu.sync_copy(esum_s.at[pl.ds(off + qo, sz)],
                            exv.at[pl.ds(0, sz)])
            pltpu.sync_copy(exv.at[pl.ds(0, sz)],
                            esum_out.at[pl.ds(off + qo, sz)])

    for qo, sz in _chunks:
        pltpu.sync_copy(rst_s.at[pl.ds(off + qo, sz), :],
                        msg.at[pl.ds(0, sz), :])

        @pl.when(c == 0)
        def _wl(qo=qo, sz=sz):
            pltpu.sync_copy(msg.at[pl.ds(0, sz), :],
                            rstL.at[pl.ds(off + qo, sz), :])

        @pl.when(c == 1)
        def _wr(qo=qo, sz=sz):
            pltpu.sync_copy(msg.at[pl.ds(0, sz), :],
                            rstR.at[pl.ds(off + qo, sz), :])


def _k3_body(rL0, rR0, rL1, rR1, rL2, rR2, e0, e1, e2,
             w1_ref, b1_ref, w2_ref, z3_ref, wacc_ref):
    ib = pl.program_id(0)
    rowi = lax.broadcasted_iota(jnp.int32, (8, 128), 0)
    acc = jnp.zeros((8, 128), jnp.float32)
    for i, (rl, rr, er) in enumerate(((rL0, rR0, e0), (rL1, rR1, e1),
                                      (rL2, rR2, e2))):
        r = jnp.concatenate([rl[...], rr[...]], axis=1)
        z = r / (er[...] + 1e-9)
        z = jnp.where(z > 0, z, jnp.exp(jnp.minimum(z, 0.0)) - 1.0)
        z3_ref[i] = z
        t = jnp.tanh(jnp.dot(z, w1_ref[...],
                             preferred_element_type=jnp.float32) + b1_ref[...])
        ssum = jnp.sum(t * w2_ref[...])
        acc = acc + jnp.where(rowi == i, ssum, 0.0)

    @pl.when(ib == 0)
    def _init():
        wacc_ref[...] = acc

    @pl.when(ib > 0)
    def _accum():
        wacc_ref[...] += acc


def _k3(rsts, esums, w1, b1r, w2r):
    args = []
    for rl, rr in rsts:
        args += [rl, rr]
    args += list(esums) + [w1, b1r, w2r]
    return pl.pallas_call(
        _k3_body,
        grid=(NB,),
        in_specs=(
            [pl.BlockSpec((BN, 16), lambda b: (b, 0))] * 6
            + [pl.BlockSpec((BN, 1), lambda b: (b, 0))] * 3
            + [pl.BlockSpec((IN, HID), lambda b: (0, 0)),
               pl.BlockSpec((1, HID), lambda b: (0, 0)),
               pl.BlockSpec((1, HID), lambda b: (0, 0))]
        ),
        out_specs=[
            pl.BlockSpec((3, BN, D), lambda b: (0, b, 0)),
            pl.BlockSpec((8, 128), lambda b: (0, 0)),
        ],
        out_shape=[
            jax.ShapeDtypeStruct((3, N, D), jnp.float32),
            jax.ShapeDtypeStruct((8, 128), jnp.float32),
        ],
    )(*args)


def _k4_body(z3_ref, wacc_ref, out_ref):
    rowi = lax.broadcasted_iota(jnp.int32, (8, 128), 0)
    t = wacc_ref[...]
    w = [jnp.sum(jnp.where(rowi == i, t, 0.0)) / (128.0 * N)
         for i in range(3)]
    m = jnp.maximum(jnp.maximum(w[0], w[1]), w[2])
    wt = jnp.where(rowi == 0, w[0], jnp.where(rowi == 1, w[1], w[2]))
    bt = jnp.exp(wt - m)
    b = [jnp.sum(jnp.where(rowi == i, bt, 0.0)) / 128.0 for i in range(3)]
    ssum = b[0] + b[1] + b[2]
    out_ref[...] = ((b[0] / ssum) * z3_ref[0] + (b[1] / ssum) * z3_ref[1]
                    + (b[2] / ssum) * z3_ref[2])


def _k4(z3, wacc):
    return pl.pallas_call(
        _k4_body,
        grid=(NB,),
        in_specs=[
            pl.BlockSpec((3, BN, D), lambda b: (0, b, 0)),
            pl.BlockSpec((8, 128), lambda b: (0, 0)),
        ],
        out_specs=pl.BlockSpec((BN, D), lambda b: (b, 0)),
        out_shape=jax.ShapeDtypeStruct((N, D), jnp.float32),
    )(z3, wacc)


def kernel(h, edge_index_0, edge_index_1, edge_index_2,
           W_0, W_1, W_2, attn_l_0, attn_l_1, attn_l_2,
           attn_r_0, attn_r_1, attn_r_2, sem_W1, sem_b1, sem_W2):
    w3 = jnp.stack([W_0, W_1, W_2])
    al3 = jnp.stack([attn_l_0, attn_l_1, attn_l_2])
    ar3 = jnp.stack([attn_r_0, attn_r_1, attn_r_2])
    f3, el3, er3 = _k1(h, w3, al3, ar3)

    rsts, esums = [], []
    for i, ei in enumerate((edge_index_0, edge_index_1, edge_index_2)):
        featT = f3[i].reshape(2 * N, 17)
        el = el3[i].reshape(N)
        erp = er3[i].reshape(N)
        csh = jnp.full((16,), jnp.max(el) + jnp.max(erp), jnp.float32)
        src2d = ei[0].reshape(E // 128, 128)
        dst2d = ei[1].reshape(E // 128, 128)
        rstL, rstR, esum = _edge_pass(featT, erp, src2d, dst2d, csh)
        rsts.append((rstL, rstR))
        esums.append(esum.reshape(NP, 1))

    z3, wacc = _k3(rsts, esums, sem_W1, sem_b1.reshape(1, HID),
                   sem_W2.reshape(1, HID))
    return _k4(z3, wacc)
